# Initial kernel scaffold; baseline (speedup 1.0000x reference)
import jax
import jax.numpy as jnp
from jax.experimental import pallas as pl


def kernel(x, edge_index, W1, b1, W2, b2, W3, b3):
    raise NotImplementedError("write your pallas kernel here")


# speedup vs baseline: 13.0894x; 13.0894x over previous
import functools

import jax
import jax.numpy as jnp
from jax import lax
from jax.experimental import pallas as pl
from jax.experimental.pallas import tpu as pltpu
from jax.experimental.pallas import tpu_sc as plsc

N_NODES = 10000
N_PAD = 10240
E = 320000
D_IN = 128
F = 64
NC, NS, L = 2, 16, 16
NW = NC * NS
E_PAD = 327680
EPW = E_PAD // NW
CHUNK = 128
NCHUNK = EPW // CHUNK
ROWS_PT = N_PAD // NS


def _sc_mesh():
    return plsc.VectorSubcoreMesh(
        core_axis_name="c", subcore_axis_name="s", num_cores=NC, num_subcores=NS
    )


DEG_R = N_PAD // CHUNK


@functools.partial(
    pl.kernel,
    out_type=jax.ShapeDtypeStruct((NC * N_PAD,), jnp.float32),
    mesh=_sc_mesh(),
    scratch_types=[
        pltpu.VMEM((NCHUNK, CHUNK), jnp.int32),
        pltpu.VMEM((N_PAD,), jnp.float32),
        pltpu.VMEM((DEG_R, CHUNK), jnp.int32),
        pltpu.VMEM_SHARED((N_PAD,), jnp.float32),
    ],
    compiler_params=pltpu.CompilerParams(needs_layout_passes=False),
)
def _sc_degree(dst_hbm, out_hbm, idx_v, acc_v, idm_v, acc_sh):
    cid = lax.axis_index("c")
    sid = lax.axis_index("s")
    wid = sid * NC + cid
    pltpu.sync_copy(dst_hbm.at[wid], idx_v)

    zero = jnp.zeros((L,), jnp.float32)
    lane = lax.iota(jnp.int32, L)

    def zbody(i, carry):
        acc_v[pl.ds(i * L, L)] = zero
        idm_v[i // (CHUNK // L), pl.ds((i % (CHUNK // L)) * L, L)] = i * L + lane
        return carry

    lax.fori_loop(0, N_PAD // L, zbody, 0)
    pltpu.sync_copy(
        acc_v.at[pl.ds(0, ROWS_PT)], acc_sh.at[pl.ds(sid * ROWS_PT, ROWS_PT)]
    )
    plsc.subcore_barrier()

    ones = jnp.ones((L,), jnp.float32)

    def body(i, carry):
        c = i // (CHUNK // L)
        j = i % (CHUNK // L)
        idx = idx_v[c, pl.ds(j * L, L)]
        plsc.addupdate_scatter(acc_v, [idx], ones)
        return carry

    lax.fori_loop(0, EPW // L, body, 0)

    def mbody(c, carry):
        pltpu.sync_copy(
            acc_v.at[pl.ds(c * CHUNK, CHUNK)], acc_sh.at[idm_v.at[c]], add=True
        )
        return carry

    lax.fori_loop(0, DEG_R, mbody, 0)
    plsc.subcore_barrier()
    pltpu.sync_copy(
        acc_sh.at[pl.ds(sid * ROWS_PT, ROWS_PT)],
        out_hbm.at[pl.ds(cid * N_PAD + sid * ROWS_PT, ROWS_PT)],
    )


@functools.partial(
    pl.kernel,
    out_type=jax.ShapeDtypeStruct((NC, N_PAD, F), jnp.float32),
    mesh=_sc_mesh(),
    scratch_types=[
        pltpu.VMEM((NCHUNK, CHUNK), jnp.int32),
        pltpu.VMEM((NCHUNK, CHUNK), jnp.int32),
        pltpu.VMEM((CHUNK, F), jnp.float32),
        pltpu.VMEM_SHARED((N_PAD, F), jnp.float32),
        pltpu.SemaphoreType.DMA,
    ],
    compiler_params=pltpu.CompilerParams(
        needs_layout_passes=False, use_tc_tiling_on_sc=False
    ),
)
def _sc_propagate(g_hbm, src_hbm, dst_hbm, zeros_hbm, out_hbm,
                  srcv, dstv, buf, acc_sh, sem):
    cid = lax.axis_index("c")
    sid = lax.axis_index("s")
    wid = sid * NC + cid
    pltpu.sync_copy(src_hbm.at[wid], srcv)
    pltpu.sync_copy(dst_hbm.at[wid], dstv)
    pltpu.sync_copy(
        zeros_hbm.at[pl.ds(sid * ROWS_PT, ROWS_PT)],
        acc_sh.at[pl.ds(sid * ROWS_PT, ROWS_PT)],
    )
    plsc.subcore_barrier()

    def body(c, carry):
        pltpu.async_copy(g_hbm.at[srcv.at[c]], buf, sem).wait()
        pltpu.sync_copy(buf, acc_sh.at[dstv.at[c]], add=True)
        return carry

    lax.fori_loop(0, NCHUNK, body, 0)
    plsc.subcore_barrier()
    pltpu.sync_copy(
        acc_sh.at[pl.ds(sid * ROWS_PT, ROWS_PT)],
        out_hbm.at[cid, pl.ds(sid * ROWS_PT, ROWS_PT)],
    )


def _tc_prep_body(x_ref, w1_ref, w2_ref, w3_ref, b1_ref, b2_ref, degp_ref,
                  g0_ref, dinv_ref, ca_ref, cb_ref):
    w23 = jnp.dot(w2_ref[...], w3_ref[...], preferred_element_type=jnp.float32)
    wc = jnp.dot(w1_ref[...], w23, preferred_element_type=jnp.float32)
    h0 = jnp.dot(x_ref[...], wc, preferred_element_type=jnp.float32)
    deg = degp_ref[0] + degp_ref[1] + 1.0
    row = lax.broadcasted_iota(jnp.int32, (N_PAD, 1), 0)
    dinv = jnp.where(row < N_NODES, lax.rsqrt(deg), 0.0)
    dinv_ref[...] = dinv
    g0_ref[...] = dinv * h0
    ca_ref[...] = jnp.dot(b1_ref[...], w23, preferred_element_type=jnp.float32)
    cb_ref[...] = jnp.dot(b2_ref[...], w3_ref[...],
                          preferred_element_type=jnp.float32)


_tc_prep = pl.pallas_call(
    _tc_prep_body,
    out_shape=(
        jax.ShapeDtypeStruct((N_PAD, F), jnp.float32),
        jax.ShapeDtypeStruct((N_PAD, 1), jnp.float32),
        jax.ShapeDtypeStruct((1, F), jnp.float32),
        jax.ShapeDtypeStruct((1, F), jnp.float32),
    ),
)


def _tc_combine_body(p_ref, g_ref, dinv_ref, c_ref, out_ref):
    t = p_ref[0] + p_ref[1] + g_ref[...]
    dinv = dinv_ref[...]
    out_ref[...] = (dinv * dinv) * t + dinv * c_ref[...]


_tc_combine = pl.pallas_call(
    _tc_combine_body,
    out_shape=jax.ShapeDtypeStruct((N_PAD, F), jnp.float32),
)


def _tc_final_body(p_ref, g_ref, dinv_ref, b3_ref, out_ref):
    t = p_ref[0] + p_ref[1] + g_ref[...]
    h = dinv_ref[...] * t + b3_ref[...]
    m = jnp.max(h, axis=1, keepdims=True)
    e = jnp.exp(h - m)
    s = jnp.sum(e, axis=1, keepdims=True)
    out_ref[...] = h - m - jnp.log(s)


_tc_final = pl.pallas_call(
    _tc_final_body,
    out_shape=jax.ShapeDtypeStruct((N_PAD, F), jnp.float32),
)


def kernel(x, edge_index, W1, b1, W2, b2, W3, b3):
    src = edge_index[0].astype(jnp.int32)
    dst = edge_index[1].astype(jnp.int32)
    pad = jnp.full((E_PAD - E,), N_NODES, jnp.int32)
    src3 = jnp.concatenate([src, pad]).reshape(NW, NCHUNK, CHUNK)
    dst3 = jnp.concatenate([dst, pad]).reshape(NW, NCHUNK, CHUNK)
    x_pad = jnp.pad(x.astype(jnp.float32), ((0, N_PAD - N_NODES), (0, 0)))
    zeros = jnp.zeros((N_PAD, F), jnp.float32)
    b1r = b1.reshape(1, -1)
    b2r = b2.reshape(1, -1)
    b3r = b3.reshape(1, -1)

    degp = _sc_degree(dst3)
    degp_col = degp.reshape(NC, N_PAD, 1)
    g0, dinv, ca, cb = _tc_prep(x_pad, W1, W2, W3, b1r, b2r, degp_col)
    p = _sc_propagate(g0, src3, dst3, zeros)
    g1 = _tc_combine(p, g0, dinv, ca)
    p = _sc_propagate(g1, src3, dst3, zeros)
    g2 = _tc_combine(p, g1, dinv, cb)
    p = _sc_propagate(g2, src3, dst3, zeros)
    out = _tc_final(p, g2, dinv, b3r)
    return out[:N_NODES]


# speedup vs baseline: 15.2569x; 1.1656x over previous
import functools

import jax
import jax.numpy as jnp
from jax import lax
from jax.experimental import pallas as pl
from jax.experimental.pallas import tpu as pltpu
from jax.experimental.pallas import tpu_sc as plsc

N_NODES = 10000
N_PAD = 10240
E = 320000
D_IN = 128
F = 64
NC, NS, L = 2, 16, 16
NW = NC * NS
E_PAD = 327680
EPW = E_PAD // NW
CHUNK = 128
NCHUNK = EPW // CHUNK
ROWS_PT = N_PAD // NS


def _sc_mesh():
    return plsc.VectorSubcoreMesh(
        core_axis_name="c", subcore_axis_name="s", num_cores=NC, num_subcores=NS
    )


DEG_R = N_PAD // CHUNK


@functools.partial(
    pl.kernel,
    out_type=jax.ShapeDtypeStruct((NC * N_PAD,), jnp.float32),
    mesh=_sc_mesh(),
    scratch_types=[
        pltpu.VMEM((NCHUNK, CHUNK), jnp.int32),
        pltpu.VMEM((N_PAD,), jnp.float32),
        pltpu.VMEM((DEG_R, CHUNK), jnp.int32),
        pltpu.VMEM_SHARED((N_PAD,), jnp.float32),
    ],
    compiler_params=pltpu.CompilerParams(needs_layout_passes=False),
)
def _sc_degree(dst_hbm, out_hbm, idx_v, acc_v, idm_v, acc_sh):
    cid = lax.axis_index("c")
    sid = lax.axis_index("s")
    wid = sid * NC + cid
    pltpu.sync_copy(dst_hbm.at[wid], idx_v)

    zero = jnp.zeros((L,), jnp.float32)
    lane = lax.iota(jnp.int32, L)

    def zbody(i, carry):
        acc_v[pl.ds(i * L, L)] = zero
        idm_v[i // (CHUNK // L), pl.ds((i % (CHUNK // L)) * L, L)] = i * L + lane
        return carry

    lax.fori_loop(0, N_PAD // L, zbody, 0)
    pltpu.sync_copy(
        acc_v.at[pl.ds(0, ROWS_PT)], acc_sh.at[pl.ds(sid * ROWS_PT, ROWS_PT)]
    )
    plsc.subcore_barrier()

    ones = jnp.ones((L,), jnp.float32)

    def body(i, carry):
        c = i // (CHUNK // L)
        j = i % (CHUNK // L)
        idx = idx_v[c, pl.ds(j * L, L)]
        plsc.addupdate_scatter(acc_v, [idx], ones)
        return carry

    lax.fori_loop(0, EPW // L, body, 0)

    def mbody(c, carry):
        pltpu.sync_copy(
            acc_v.at[pl.ds(c * CHUNK, CHUNK)], acc_sh.at[idm_v.at[c]], add=True
        )
        return carry

    lax.fori_loop(0, DEG_R, mbody, 0)
    plsc.subcore_barrier()
    pltpu.sync_copy(
        acc_sh.at[pl.ds(sid * ROWS_PT, ROWS_PT)],
        out_hbm.at[pl.ds(cid * N_PAD + sid * ROWS_PT, ROWS_PT)],
    )


NBUF = 4


@functools.partial(
    pl.kernel,
    out_type=jax.ShapeDtypeStruct((NC, N_PAD, F), jnp.float32),
    mesh=_sc_mesh(),
    scratch_types=[
        pltpu.VMEM((NCHUNK, CHUNK), jnp.int32),
        pltpu.VMEM((NCHUNK, CHUNK), jnp.int32),
        pltpu.VMEM((NBUF, CHUNK, F), jnp.float32),
        pltpu.VMEM_SHARED((N_PAD, F), jnp.float32),
    ]
    + [pltpu.SemaphoreType.DMA] * (2 * NBUF),
    compiler_params=pltpu.CompilerParams(
        needs_layout_passes=False, use_tc_tiling_on_sc=False
    ),
)
def _sc_propagate(g_hbm, src_hbm, dst_hbm, zeros_hbm, out_hbm,
                  srcv, dstv, buf, acc_sh, *sems):
    semg = sems[:NBUF]
    sems_ = sems[NBUF:]
    cid = lax.axis_index("c")
    sid = lax.axis_index("s")
    wid = sid * NC + cid
    pltpu.sync_copy(src_hbm.at[wid], srcv)
    pltpu.sync_copy(dst_hbm.at[wid], dstv)
    pltpu.sync_copy(
        zeros_hbm.at[pl.ds(sid * ROWS_PT, ROWS_PT)],
        acc_sh.at[pl.ds(sid * ROWS_PT, ROWS_PT)],
    )
    plsc.subcore_barrier()

    def gather_start(c, b):
        pltpu.async_copy(g_hbm.at[srcv.at[c]], buf.at[b], semg[b])

    def gather_wait(c, b):
        pltpu.make_async_copy(g_hbm.at[srcv.at[c]], buf.at[b], semg[b]).wait()

    def scatter_start(c, b):
        pltpu.async_copy(buf.at[b], acc_sh.at[dstv.at[c]], sems_[b], add=True)

    def scatter_wait(c, b):
        pltpu.make_async_copy(buf.at[b], acc_sh.at[dstv.at[c]], sems_[b]).wait()

    for b in range(NBUF):
        gather_start(b, b)

    def body(i, carry):
        base = i * NBUF
        for b in range(NBUF):
            gather_wait(base + b, b)
            scatter_start(base + b, b)
        for b in range(NBUF):
            scatter_wait(base + b, b)
            gather_start(base + NBUF + b, b)
        return carry

    lax.fori_loop(0, NCHUNK // NBUF - 1, body, 0)
    last = NCHUNK - NBUF
    for b in range(NBUF):
        gather_wait(last + b, b)
        scatter_start(last + b, b)
    for b in range(NBUF):
        scatter_wait(last + b, b)
    plsc.subcore_barrier()
    pltpu.sync_copy(
        acc_sh.at[pl.ds(sid * ROWS_PT, ROWS_PT)],
        out_hbm.at[cid, pl.ds(sid * ROWS_PT, ROWS_PT)],
    )


def _tc_prep_body(x_ref, w1_ref, w2_ref, w3_ref, b1_ref, b2_ref, degp_ref,
                  g0_ref, dinv_ref, ca_ref, cb_ref):
    w23 = jnp.dot(w2_ref[...], w3_ref[...], preferred_element_type=jnp.float32)
    wc = jnp.dot(w1_ref[...], w23, preferred_element_type=jnp.float32)
    h0 = jnp.dot(x_ref[...], wc, preferred_element_type=jnp.float32)
    deg = degp_ref[0] + degp_ref[1] + 1.0
    row = lax.broadcasted_iota(jnp.int32, (N_PAD, 1), 0)
    dinv = jnp.where(row < N_NODES, lax.rsqrt(deg), 0.0)
    dinv_ref[...] = dinv
    g0_ref[...] = dinv * h0
    ca_ref[...] = jnp.dot(b1_ref[...], w23, preferred_element_type=jnp.float32)
    cb_ref[...] = jnp.dot(b2_ref[...], w3_ref[...],
                          preferred_element_type=jnp.float32)


_tc_prep = pl.pallas_call(
    _tc_prep_body,
    out_shape=(
        jax.ShapeDtypeStruct((N_PAD, F), jnp.float32),
        jax.ShapeDtypeStruct((N_PAD, 1), jnp.float32),
        jax.ShapeDtypeStruct((1, F), jnp.float32),
        jax.ShapeDtypeStruct((1, F), jnp.float32),
    ),
)


def _tc_combine_body(p_ref, g_ref, dinv_ref, c_ref, out_ref):
    t = p_ref[0] + p_ref[1] + g_ref[...]
    dinv = dinv_ref[...]
    out_ref[...] = (dinv * dinv) * t + dinv * c_ref[...]


_tc_combine = pl.pallas_call(
    _tc_combine_body,
    out_shape=jax.ShapeDtypeStruct((N_PAD, F), jnp.float32),
)


def _tc_final_body(p_ref, g_ref, dinv_ref, b3_ref, out_ref):
    t = p_ref[0] + p_ref[1] + g_ref[...]
    h = dinv_ref[...] * t + b3_ref[...]
    m = jnp.max(h, axis=1, keepdims=True)
    e = jnp.exp(h - m)
    s = jnp.sum(e, axis=1, keepdims=True)
    out_ref[...] = h - m - jnp.log(s)


_tc_final = pl.pallas_call(
    _tc_final_body,
    out_shape=jax.ShapeDtypeStruct((N_PAD, F), jnp.float32),
)


def kernel(x, edge_index, W1, b1, W2, b2, W3, b3):
    src = edge_index[0].astype(jnp.int32)
    dst = edge_index[1].astype(jnp.int32)
    pad = jnp.full((E_PAD - E,), N_NODES, jnp.int32)
    src3 = jnp.concatenate([src, pad]).reshape(NW, NCHUNK, CHUNK)
    dst3 = jnp.concatenate([dst, pad]).reshape(NW, NCHUNK, CHUNK)
    x_pad = jnp.pad(x.astype(jnp.float32), ((0, N_PAD - N_NODES), (0, 0)))
    zeros = jnp.zeros((N_PAD, F), jnp.float32)
    b1r = b1.reshape(1, -1)
    b2r = b2.reshape(1, -1)
    b3r = b3.reshape(1, -1)

    degp = _sc_degree(dst3)
    degp_col = degp.reshape(NC, N_PAD, 1)
    g0, dinv, ca, cb = _tc_prep(x_pad, W1, W2, W3, b1r, b2r, degp_col)
    p = _sc_propagate(g0, src3, dst3, zeros)
    g1 = _tc_combine(p, g0, dinv, ca)
    p = _sc_propagate(g1, src3, dst3, zeros)
    g2 = _tc_combine(p, g1, dinv, cb)
    p = _sc_propagate(g2, src3, dst3, zeros)
    out = _tc_final(p, g2, dinv, b3r)
    return out[:N_NODES]


# speedup vs baseline: 16.4324x; 1.0770x over previous
import functools

import jax
import jax.numpy as jnp
from jax import lax
from jax.experimental import pallas as pl
from jax.experimental.pallas import tpu as pltpu
from jax.experimental.pallas import tpu_sc as plsc

N_NODES = 10000
N_PAD = 10240
E = 320000
D_IN = 128
F = 64
NC, NS, L = 2, 16, 16
NW = NC * NS
E_PAD = 327680
EPW = E_PAD // NW
CHUNK = 128
NCHUNK = EPW // CHUNK
ROWS_PT = N_PAD // NS


def _sc_mesh():
    return plsc.VectorSubcoreMesh(
        core_axis_name="c", subcore_axis_name="s", num_cores=NC, num_subcores=NS
    )


DEG_R = N_PAD // CHUNK


@functools.partial(
    pl.kernel,
    out_type=jax.ShapeDtypeStruct((NC * N_PAD,), jnp.float32),
    mesh=_sc_mesh(),
    scratch_types=[
        pltpu.VMEM((NCHUNK, CHUNK), jnp.int32),
        pltpu.VMEM((N_PAD,), jnp.float32),
        pltpu.VMEM((DEG_R, CHUNK), jnp.int32),
        pltpu.VMEM_SHARED((N_PAD,), jnp.float32),
    ],
    compiler_params=pltpu.CompilerParams(needs_layout_passes=False),
)
def _sc_degree(dst_hbm, out_hbm, idx_v, acc_v, idm_v, acc_sh):
    cid = lax.axis_index("c")
    sid = lax.axis_index("s")
    wid = sid * NC + cid
    pltpu.sync_copy(dst_hbm.at[pl.ds(wid * NCHUNK, NCHUNK)], idx_v)

    zero = jnp.zeros((L,), jnp.float32)
    lane = lax.iota(jnp.int32, L)

    def zbody(i, carry):
        acc_v[pl.ds(i * L, L)] = zero
        idm_v[i // (CHUNK // L), pl.ds((i % (CHUNK // L)) * L, L)] = i * L + lane
        return carry

    lax.fori_loop(0, N_PAD // L, zbody, 0)
    pltpu.sync_copy(
        acc_v.at[pl.ds(0, ROWS_PT)], acc_sh.at[pl.ds(sid * ROWS_PT, ROWS_PT)]
    )
    plsc.subcore_barrier()

    ones = jnp.ones((L,), jnp.float32)

    def body(i, carry):
        c = i // (CHUNK // L)
        j = i % (CHUNK // L)
        idx = idx_v[c, pl.ds(j * L, L)]
        plsc.addupdate_scatter(acc_v, [idx], ones)
        return carry

    lax.fori_loop(0, EPW // L, body, 0)

    def mbody(c, carry):
        pltpu.sync_copy(
            acc_v.at[pl.ds(c * CHUNK, CHUNK)], acc_sh.at[idm_v.at[c]], add=True
        )
        return carry

    lax.fori_loop(0, DEG_R, mbody, 0)
    plsc.subcore_barrier()
    pltpu.sync_copy(
        acc_sh.at[pl.ds(sid * ROWS_PT, ROWS_PT)],
        out_hbm.at[pl.ds(cid * N_PAD + sid * ROWS_PT, ROWS_PT)],
    )


NBUF = 4
A_CH = 120
B_CH = 40


@functools.partial(
    pl.kernel,
    out_type=jax.ShapeDtypeStruct((NC, N_PAD, F), jnp.float32),
    mesh=_sc_mesh(),
    scratch_types=[
        pltpu.VMEM((A_CH, CHUNK), jnp.int32),
        pltpu.VMEM((A_CH, CHUNK), jnp.int32),
        pltpu.VMEM((NBUF, CHUNK, F), jnp.float32),
        pltpu.VMEM_SHARED((N_PAD, F), jnp.float32),
    ]
    + [pltpu.SemaphoreType.DMA] * (2 * NBUF),
    compiler_params=pltpu.CompilerParams(
        needs_layout_passes=False, use_tc_tiling_on_sc=False
    ),
)
def _sc_propagate(g_hbm, src_hbm, dst_hbm, zeros_hbm, out_hbm,
                  srcv, dstv, buf, acc_sh, *sems):
    semg = sems[:NBUF]
    sems_ = sems[NBUF:]
    cid = lax.axis_index("c")
    sid = lax.axis_index("s")
    pltpu.sync_copy(
        zeros_hbm.at[pl.ds(sid * ROWS_PT, ROWS_PT)],
        acc_sh.at[pl.ds(sid * ROWS_PT, ROWS_PT)],
    )

    def gather_start(c, b):
        pltpu.async_copy(g_hbm.at[srcv.at[c]], buf.at[b], semg[b])

    def gather_wait(c, b):
        pltpu.make_async_copy(g_hbm.at[srcv.at[c]], buf.at[b], semg[b]).wait()

    def scatter_start(c, b):
        pltpu.async_copy(buf.at[b], acc_sh.at[dstv.at[c]], sems_[b], add=True)

    def scatter_wait(c, b):
        pltpu.make_async_copy(buf.at[b], acc_sh.at[dstv.at[c]], sems_[b]).wait()

    def run(nch, rowbase):
        pltpu.sync_copy(
            src_hbm.at[pl.ds(rowbase, nch)], srcv.at[pl.ds(0, nch)]
        )
        pltpu.sync_copy(
            dst_hbm.at[pl.ds(rowbase, nch)], dstv.at[pl.ds(0, nch)]
        )
        plsc.subcore_barrier()
        for b in range(NBUF):
            gather_start(b, b)

        def body(i, carry):
            base = i * NBUF
            for b in range(NBUF):
                gather_wait(base + b, b)
                scatter_start(base + b, b)
            for b in range(NBUF):
                scatter_wait(base + b, b)
                gather_start(base + NBUF + b, b)
            return carry

        lax.fori_loop(0, nch // NBUF - 1, body, 0)
        last = nch - NBUF
        for b in range(NBUF):
            gather_wait(last + b, b)
            scatter_start(last + b, b)
        for b in range(NBUF):
            scatter_wait(last + b, b)

    @pl.when(cid == 0)
    def _():
        run(A_CH, sid * (A_CH + B_CH))

    @pl.when(cid == 1)
    def _():
        run(B_CH, sid * (A_CH + B_CH) + A_CH)

    plsc.subcore_barrier()
    pltpu.sync_copy(
        acc_sh.at[pl.ds(sid * ROWS_PT, ROWS_PT)],
        out_hbm.at[cid, pl.ds(sid * ROWS_PT, ROWS_PT)],
    )


def _tc_prep_body(x_ref, w1_ref, w2_ref, w3_ref, b1_ref, b2_ref, degp_ref,
                  g0_ref, dinv_ref, ca_ref, cb_ref):
    w23 = jnp.dot(w2_ref[...], w3_ref[...], preferred_element_type=jnp.float32)
    wc = jnp.dot(w1_ref[...], w23, preferred_element_type=jnp.float32)
    h0 = jnp.dot(x_ref[...], wc, preferred_element_type=jnp.float32)
    deg = degp_ref[0] + degp_ref[1] + 1.0
    row = lax.broadcasted_iota(jnp.int32, (N_PAD, 1), 0)
    dinv = jnp.where(row < N_NODES, lax.rsqrt(deg), 0.0)
    dinv_ref[...] = dinv
    g0_ref[...] = dinv * h0
    ca_ref[...] = jnp.dot(b1_ref[...], w23, preferred_element_type=jnp.float32)
    cb_ref[...] = jnp.dot(b2_ref[...], w3_ref[...],
                          preferred_element_type=jnp.float32)


_tc_prep = pl.pallas_call(
    _tc_prep_body,
    out_shape=(
        jax.ShapeDtypeStruct((N_PAD, F), jnp.float32),
        jax.ShapeDtypeStruct((N_PAD, 1), jnp.float32),
        jax.ShapeDtypeStruct((1, F), jnp.float32),
        jax.ShapeDtypeStruct((1, F), jnp.float32),
    ),
)


def _tc_combine_body(p_ref, g_ref, dinv_ref, c_ref, out_ref):
    t = p_ref[0] + p_ref[1] + g_ref[...]
    dinv = dinv_ref[...]
    out_ref[...] = (dinv * dinv) * t + dinv * c_ref[...]


_tc_combine = pl.pallas_call(
    _tc_combine_body,
    out_shape=jax.ShapeDtypeStruct((N_PAD, F), jnp.float32),
)


def _tc_final_body(p_ref, g_ref, dinv_ref, b3_ref, out_ref):
    t = p_ref[0] + p_ref[1] + g_ref[...]
    h = dinv_ref[...] * t + b3_ref[...]
    m = jnp.max(h, axis=1, keepdims=True)
    e = jnp.exp(h - m)
    s = jnp.sum(e, axis=1, keepdims=True)
    out_ref[...] = h - m - jnp.log(s)


_tc_final = pl.pallas_call(
    _tc_final_body,
    out_shape=jax.ShapeDtypeStruct((N_PAD, F), jnp.float32),
)


def kernel(x, edge_index, W1, b1, W2, b2, W3, b3):
    src = edge_index[0].astype(jnp.int32)
    dst = edge_index[1].astype(jnp.int32)
    pad = jnp.full((E_PAD - E,), N_NODES, jnp.int32)
    src3 = jnp.concatenate([src, pad]).reshape(E_PAD // CHUNK, CHUNK)
    dst3 = jnp.concatenate([dst, pad]).reshape(E_PAD // CHUNK, CHUNK)
    x_pad = jnp.pad(x.astype(jnp.float32), ((0, N_PAD - N_NODES), (0, 0)))
    zeros = jnp.zeros((N_PAD, F), jnp.float32)
    b1r = b1.reshape(1, -1)
    b2r = b2.reshape(1, -1)
    b3r = b3.reshape(1, -1)

    degp = _sc_degree(dst3)
    degp_col = degp.reshape(NC, N_PAD, 1)
    g0, dinv, ca, cb = _tc_prep(x_pad, W1, W2, W3, b1r, b2r, degp_col)
    p = _sc_propagate(g0, src3, dst3, zeros)
    g1 = _tc_combine(p, g0, dinv, ca)
    p = _sc_propagate(g1, src3, dst3, zeros)
    g2 = _tc_combine(p, g1, dinv, cb)
    p = _sc_propagate(g2, src3, dst3, zeros)
    out = _tc_final(p, g2, dinv, b3r)
    return out[:N_NODES]


# speedup vs baseline: 35.5743x; 2.1649x over previous
import functools

import jax
import jax.numpy as jnp
from jax import lax
from jax.experimental import pallas as pl
from jax.experimental.pallas import tpu as pltpu
from jax.experimental.pallas import tpu_sc as plsc

N_NODES = 10000
N_PAD = 10240
E = 320000
D_IN = 128
F = 64
NC, NS, L = 2, 16, 16
NW = NC * NS
E_PAD = 327680
EPW = E_PAD // NW
CHUNK = 128
NCHUNK = EPW // CHUNK
ROWS_PT = N_PAD // NS


def _sc_mesh():
    return plsc.VectorSubcoreMesh(
        core_axis_name="c", subcore_axis_name="s", num_cores=NC, num_subcores=NS
    )


DEG_R = N_PAD // CHUNK


@functools.partial(
    pl.kernel,
    out_type=jax.ShapeDtypeStruct((NC * N_PAD,), jnp.float32),
    mesh=_sc_mesh(),
    scratch_types=[
        pltpu.VMEM((NCHUNK, CHUNK), jnp.int32),
        pltpu.VMEM((N_PAD,), jnp.float32),
        pltpu.VMEM((DEG_R, CHUNK), jnp.int32),
        pltpu.VMEM_SHARED((N_PAD,), jnp.float32),
    ],
    compiler_params=pltpu.CompilerParams(needs_layout_passes=False),
)
def _sc_degree(dst_hbm, out_hbm, idx_v, acc_v, idm_v, acc_sh):
    cid = lax.axis_index("c")
    sid = lax.axis_index("s")
    wid = sid * NC + cid
    pltpu.sync_copy(dst_hbm.at[pl.ds(wid * NCHUNK, NCHUNK)], idx_v)

    zero = jnp.zeros((L,), jnp.float32)
    lane = lax.iota(jnp.int32, L)

    def zbody(i, carry):
        acc_v[pl.ds(i * L, L)] = zero
        idm_v[i // (CHUNK // L), pl.ds((i % (CHUNK // L)) * L, L)] = i * L + lane
        return carry

    lax.fori_loop(0, N_PAD // L, zbody, 0)
    pltpu.sync_copy(
        acc_v.at[pl.ds(0, ROWS_PT)], acc_sh.at[pl.ds(sid * ROWS_PT, ROWS_PT)]
    )
    plsc.subcore_barrier()

    ones = jnp.ones((L,), jnp.float32)

    def body(i, carry):
        c = i // (CHUNK // L)
        j = i % (CHUNK // L)
        idx = idx_v[c, pl.ds(j * L, L)]
        plsc.addupdate_scatter(acc_v, [idx], ones)
        return carry

    lax.fori_loop(0, EPW // L, body, 0)

    def mbody(c, carry):
        pltpu.sync_copy(
            acc_v.at[pl.ds(c * CHUNK, CHUNK)], acc_sh.at[idm_v.at[c]], add=True
        )
        return carry

    lax.fori_loop(0, DEG_R, mbody, 0)
    plsc.subcore_barrier()
    pltpu.sync_copy(
        acc_sh.at[pl.ds(sid * ROWS_PT, ROWS_PT)],
        out_hbm.at[pl.ds(cid * N_PAD + sid * ROWS_PT, ROWS_PT)],
    )


NBUF = 4
A_CH = 80
B_CH = 80


@functools.partial(
    pl.kernel,
    out_type=jax.ShapeDtypeStruct((NC, N_PAD, F), jnp.float32),
    mesh=_sc_mesh(),
    scratch_types=[
        pltpu.VMEM((A_CH, CHUNK), jnp.int32),
        pltpu.VMEM((A_CH, CHUNK), jnp.int32),
        pltpu.VMEM((NBUF, CHUNK, F), jnp.float32),
        pltpu.VMEM_SHARED((N_PAD, F), jnp.float32),
    ]
    + [pltpu.SemaphoreType.DMA] * (2 * NBUF),
    compiler_params=pltpu.CompilerParams(
        needs_layout_passes=False, use_tc_tiling_on_sc=False
    ),
)
def _sc_propagate(g_hbm, src_hbm, dst_hbm, zeros_hbm, out_hbm,
                  srcv, dstv, buf, acc_sh, *sems):
    semg = sems[:NBUF]
    sems_ = sems[NBUF:]
    cid = lax.axis_index("c")
    sid = lax.axis_index("s")
    pltpu.sync_copy(
        zeros_hbm.at[pl.ds(sid * ROWS_PT, ROWS_PT)],
        acc_sh.at[pl.ds(sid * ROWS_PT, ROWS_PT)],
    )

    def gather_start(c, b):
        pltpu.async_copy(g_hbm.at[srcv.at[c]], buf.at[b], semg[b])

    def gather_wait(c, b):
        pltpu.make_async_copy(g_hbm.at[srcv.at[c]], buf.at[b], semg[b]).wait()

    def scatter_start(c, b):
        pltpu.async_copy(buf.at[b], acc_sh.at[dstv.at[c]], sems_[b], add=True)

    def scatter_wait(c, b):
        pltpu.make_async_copy(buf.at[b], acc_sh.at[dstv.at[c]], sems_[b]).wait()

    def run(nch, rowbase):
        pltpu.sync_copy(
            src_hbm.at[pl.ds(rowbase, nch)], srcv.at[pl.ds(0, nch)]
        )
        pltpu.sync_copy(
            dst_hbm.at[pl.ds(rowbase, nch)], dstv.at[pl.ds(0, nch)]
        )
        plsc.subcore_barrier()
        for b in range(NBUF):
            gather_start(b, b)

        def body(i, carry):
            base = i * NBUF
            for b in range(NBUF):
                gather_wait(base + b, b)
                scatter_start(base + b, b)
            for b in range(NBUF):
                scatter_wait(base + b, b)
                gather_start(base + NBUF + b, b)
            return carry

        lax.fori_loop(0, nch // NBUF - 1, body, 0)
        last = nch - NBUF
        for b in range(NBUF):
            gather_wait(last + b, b)
            scatter_start(last + b, b)
        for b in range(NBUF):
            scatter_wait(last + b, b)

    @pl.when(cid == 0)
    def _():
        run(A_CH, sid * (A_CH + B_CH))

    @pl.when(cid == 1)
    def _():
        run(B_CH, sid * (A_CH + B_CH) + A_CH)

    plsc.subcore_barrier()
    pltpu.sync_copy(
        acc_sh.at[pl.ds(sid * ROWS_PT, ROWS_PT)],
        out_hbm.at[cid, pl.ds(sid * ROWS_PT, ROWS_PT)],
    )


def _tc_prep_body(x_ref, w1_ref, w2_ref, w3_ref, b1_ref, b2_ref, degp_ref,
                  g0_ref, dinv_ref, ca_ref, cb_ref):
    w23 = jnp.dot(w2_ref[...], w3_ref[...], preferred_element_type=jnp.float32)
    wc = jnp.dot(w1_ref[...], w23, preferred_element_type=jnp.float32)
    h0 = jnp.dot(x_ref[...], wc, preferred_element_type=jnp.float32)
    deg = degp_ref[0] + degp_ref[1] + 1.0
    row = lax.broadcasted_iota(jnp.int32, (N_PAD, 1), 0)
    dinv = jnp.where(row < N_NODES, lax.rsqrt(deg), 0.0)
    dinv_ref[...] = dinv
    g0_ref[...] = dinv * h0
    ca_ref[...] = jnp.dot(b1_ref[...], w23, preferred_element_type=jnp.float32)
    cb_ref[...] = jnp.dot(b2_ref[...], w3_ref[...],
                          preferred_element_type=jnp.float32)


_tc_prep = pl.pallas_call(
    _tc_prep_body,
    out_shape=(
        jax.ShapeDtypeStruct((N_PAD, F), jnp.float32),
        jax.ShapeDtypeStruct((N_PAD, 1), jnp.float32),
        jax.ShapeDtypeStruct((1, F), jnp.float32),
        jax.ShapeDtypeStruct((1, F), jnp.float32),
    ),
)


def _tc_combine_body(p_ref, g_ref, dinv_ref, c_ref, out_ref):
    t = p_ref[0] + p_ref[1] + g_ref[...]
    dinv = dinv_ref[...]
    out_ref[...] = (dinv * dinv) * t + dinv * c_ref[...]


_tc_combine = pl.pallas_call(
    _tc_combine_body,
    out_shape=jax.ShapeDtypeStruct((N_PAD, F), jnp.float32),
)


def _tc_final_body(p_ref, g_ref, dinv_ref, b3_ref, out_ref):
    t = p_ref[0] + p_ref[1] + g_ref[...]
    h = dinv_ref[...] * t + b3_ref[...]
    m = jnp.max(h, axis=1, keepdims=True)
    e = jnp.exp(h - m)
    s = jnp.sum(e, axis=1, keepdims=True)
    out_ref[...] = h - m - jnp.log(s)


_tc_final = pl.pallas_call(
    _tc_final_body,
    out_shape=jax.ShapeDtypeStruct((N_PAD, F), jnp.float32),
)


def kernel(x, edge_index, W1, b1, W2, b2, W3, b3):
    src = edge_index[0].astype(jnp.int32)
    dst = edge_index[1].astype(jnp.int32)
    npad = E_PAD - E
    src_pad = N_NODES + jnp.arange(npad, dtype=jnp.int32) % (N_PAD - N_NODES)
    dst_pad_prop = jnp.arange(npad, dtype=jnp.int32) % N_PAD
    dst_pad_deg = src_pad
    src3 = jnp.concatenate([src, src_pad]).reshape(E_PAD // CHUNK, CHUNK)
    dst3 = jnp.concatenate([dst, dst_pad_prop]).reshape(E_PAD // CHUNK, CHUNK)
    dstd = jnp.concatenate([dst, dst_pad_deg]).reshape(E_PAD // CHUNK, CHUNK)
    x_pad = jnp.pad(x.astype(jnp.float32), ((0, N_PAD - N_NODES), (0, 0)))
    zeros = jnp.zeros((N_PAD, F), jnp.float32)
    b1r = b1.reshape(1, -1)
    b2r = b2.reshape(1, -1)
    b3r = b3.reshape(1, -1)

    degp = _sc_degree(dstd)
    degp_col = degp.reshape(NC, N_PAD, 1)
    g0, dinv, ca, cb = _tc_prep(x_pad, W1, W2, W3, b1r, b2r, degp_col)
    p = _sc_propagate(g0, src3, dst3, zeros)
    g1 = _tc_combine(p, g0, dinv, ca)
    p = _sc_propagate(g1, src3, dst3, zeros)
    g2 = _tc_combine(p, g1, dinv, cb)
    p = _sc_propagate(g2, src3, dst3, zeros)
    out = _tc_final(p, g2, dinv, b3r)
    return out[:N_NODES]


# speedup vs baseline: 36.7330x; 1.0326x over previous
import functools

import jax
import jax.numpy as jnp
from jax import lax
from jax.experimental import pallas as pl
from jax.experimental.pallas import tpu as pltpu
from jax.experimental.pallas import tpu_sc as plsc

N_NODES = 10000
N_PAD = 10240
E = 320000
D_IN = 128
F = 64
NC, NS, L = 2, 16, 16
NW = NC * NS
E_PAD = 327680
EPW = E_PAD // NW
CHUNK = 128
NCHUNK = EPW // CHUNK
ROWS_PT = N_PAD // NS


def _sc_mesh():
    return plsc.VectorSubcoreMesh(
        core_axis_name="c", subcore_axis_name="s", num_cores=NC, num_subcores=NS
    )


DEG_R = N_PAD // CHUNK


@functools.partial(
    pl.kernel,
    out_type=jax.ShapeDtypeStruct((NC * N_PAD,), jnp.float32),
    mesh=_sc_mesh(),
    scratch_types=[
        pltpu.VMEM((NCHUNK, CHUNK), jnp.int32),
        pltpu.VMEM((N_PAD,), jnp.float32),
        pltpu.VMEM((DEG_R, CHUNK), jnp.int32),
        pltpu.VMEM_SHARED((N_PAD,), jnp.float32),
    ],
    compiler_params=pltpu.CompilerParams(needs_layout_passes=False),
)
def _sc_degree(dst_hbm, out_hbm, idx_v, acc_v, idm_v, acc_sh):
    cid = lax.axis_index("c")
    sid = lax.axis_index("s")
    wid = sid * NC + cid
    pltpu.sync_copy(dst_hbm.at[pl.ds(wid * NCHUNK, NCHUNK)], idx_v)

    zero = jnp.zeros((L,), jnp.float32)
    lane = lax.iota(jnp.int32, L)

    def zbody(i, carry):
        acc_v[pl.ds(i * L, L)] = zero
        idm_v[i // (CHUNK // L), pl.ds((i % (CHUNK // L)) * L, L)] = i * L + lane
        return carry

    lax.fori_loop(0, N_PAD // L, zbody, 0)
    pltpu.sync_copy(
        acc_v.at[pl.ds(0, ROWS_PT)], acc_sh.at[pl.ds(sid * ROWS_PT, ROWS_PT)]
    )
    plsc.subcore_barrier()

    ones = jnp.ones((L,), jnp.float32)

    def body(i, carry):
        c = i // (CHUNK // L)
        j = i % (CHUNK // L)
        idx = idx_v[c, pl.ds(j * L, L)]
        plsc.addupdate_scatter(acc_v, [idx], ones)
        return carry

    lax.fori_loop(0, EPW // L, body, 0)

    def mbody(c, carry):
        pltpu.sync_copy(
            acc_v.at[pl.ds(c * CHUNK, CHUNK)], acc_sh.at[idm_v.at[c]], add=True
        )
        return carry

    lax.fori_loop(0, DEG_R, mbody, 0)
    plsc.subcore_barrier()
    pltpu.sync_copy(
        acc_sh.at[pl.ds(sid * ROWS_PT, ROWS_PT)],
        out_hbm.at[pl.ds(cid * N_PAD + sid * ROWS_PT, ROWS_PT)],
    )


NBUF = 8
A_CH = 80
B_CH = 80


@functools.partial(
    pl.kernel,
    out_type=jax.ShapeDtypeStruct((NC, N_PAD, F), jnp.float32),
    mesh=_sc_mesh(),
    scratch_types=[
        pltpu.VMEM((A_CH, CHUNK), jnp.int32),
        pltpu.VMEM((A_CH, CHUNK), jnp.int32),
        pltpu.VMEM((NBUF, CHUNK, F), jnp.float32),
        pltpu.VMEM_SHARED((N_PAD, F), jnp.float32),
    ]
    + [pltpu.SemaphoreType.DMA] * (2 * NBUF),
    compiler_params=pltpu.CompilerParams(
        needs_layout_passes=False, use_tc_tiling_on_sc=False
    ),
)
def _sc_propagate(g_hbm, src_hbm, dst_hbm, zeros_hbm, out_hbm,
                  srcv, dstv, buf, acc_sh, *sems):
    semg = sems[:NBUF]
    sems_ = sems[NBUF:]
    cid = lax.axis_index("c")
    sid = lax.axis_index("s")
    pltpu.sync_copy(
        zeros_hbm.at[pl.ds(sid * ROWS_PT, ROWS_PT)],
        acc_sh.at[pl.ds(sid * ROWS_PT, ROWS_PT)],
    )

    def gather_start(c, b):
        pltpu.async_copy(g_hbm.at[srcv.at[c]], buf.at[b], semg[b])

    def gather_wait(c, b):
        pltpu.make_async_copy(g_hbm.at[srcv.at[c]], buf.at[b], semg[b]).wait()

    def scatter_start(c, b):
        pltpu.async_copy(buf.at[b], acc_sh.at[dstv.at[c]], sems_[b], add=True)

    def scatter_wait(c, b):
        pltpu.make_async_copy(buf.at[b], acc_sh.at[dstv.at[c]], sems_[b]).wait()

    def run(nch, rowbase):
        pltpu.sync_copy(
            src_hbm.at[pl.ds(rowbase, nch)], srcv.at[pl.ds(0, nch)]
        )
        pltpu.sync_copy(
            dst_hbm.at[pl.ds(rowbase, nch)], dstv.at[pl.ds(0, nch)]
        )
        plsc.subcore_barrier()
        for b in range(NBUF):
            gather_start(b, b)

        def body(i, carry):
            base = i * NBUF
            for b in range(NBUF):
                gather_wait(base + b, b)
                scatter_start(base + b, b)
            for b in range(NBUF):
                scatter_wait(base + b, b)
                gather_start(base + NBUF + b, b)
            return carry

        lax.fori_loop(0, nch // NBUF - 1, body, 0)
        last = nch - NBUF
        for b in range(NBUF):
            gather_wait(last + b, b)
            scatter_start(last + b, b)
        for b in range(NBUF):
            scatter_wait(last + b, b)

    @pl.when(cid == 0)
    def _():
        run(A_CH, sid * (A_CH + B_CH))

    @pl.when(cid == 1)
    def _():
        run(B_CH, sid * (A_CH + B_CH) + A_CH)

    plsc.subcore_barrier()
    pltpu.sync_copy(
        acc_sh.at[pl.ds(sid * ROWS_PT, ROWS_PT)],
        out_hbm.at[cid, pl.ds(sid * ROWS_PT, ROWS_PT)],
    )


def _tc_prep_body(x_ref, w1_ref, w2_ref, w3_ref, b1_ref, b2_ref, degp_ref,
                  g0_ref, dinv_ref, ca_ref, cb_ref):
    w23 = jnp.dot(w2_ref[...], w3_ref[...], preferred_element_type=jnp.float32)
    wc = jnp.dot(w1_ref[...], w23, preferred_element_type=jnp.float32)
    h0 = jnp.dot(x_ref[...], wc, preferred_element_type=jnp.float32)
    deg = degp_ref[0] + degp_ref[1] + 1.0
    row = lax.broadcasted_iota(jnp.int32, (N_PAD, 1), 0)
    dinv = jnp.where(row < N_NODES, lax.rsqrt(deg), 0.0)
    dinv_ref[...] = dinv
    g0_ref[...] = dinv * h0
    ca_ref[...] = jnp.dot(b1_ref[...], w23, preferred_element_type=jnp.float32)
    cb_ref[...] = jnp.dot(b2_ref[...], w3_ref[...],
                          preferred_element_type=jnp.float32)


_tc_prep = pl.pallas_call(
    _tc_prep_body,
    out_shape=(
        jax.ShapeDtypeStruct((N_PAD, F), jnp.float32),
        jax.ShapeDtypeStruct((N_PAD, 1), jnp.float32),
        jax.ShapeDtypeStruct((1, F), jnp.float32),
        jax.ShapeDtypeStruct((1, F), jnp.float32),
    ),
)


def _tc_combine_body(p_ref, g_ref, dinv_ref, c_ref, out_ref):
    t = p_ref[0] + p_ref[1] + g_ref[...]
    dinv = dinv_ref[...]
    out_ref[...] = (dinv * dinv) * t + dinv * c_ref[...]


_tc_combine = pl.pallas_call(
    _tc_combine_body,
    out_shape=jax.ShapeDtypeStruct((N_PAD, F), jnp.float32),
)


def _tc_final_body(p_ref, g_ref, dinv_ref, b3_ref, out_ref):
    t = p_ref[0] + p_ref[1] + g_ref[...]
    h = dinv_ref[...] * t + b3_ref[...]
    m = jnp.max(h, axis=1, keepdims=True)
    e = jnp.exp(h - m)
    s = jnp.sum(e, axis=1, keepdims=True)
    out_ref[...] = h - m - jnp.log(s)


_tc_final = pl.pallas_call(
    _tc_final_body,
    out_shape=jax.ShapeDtypeStruct((N_PAD, F), jnp.float32),
)


def kernel(x, edge_index, W1, b1, W2, b2, W3, b3):
    src = edge_index[0].astype(jnp.int32)
    dst = edge_index[1].astype(jnp.int32)
    npad = E_PAD - E
    src_pad = N_NODES + jnp.arange(npad, dtype=jnp.int32) % (N_PAD - N_NODES)
    dst_pad_prop = jnp.arange(npad, dtype=jnp.int32) % N_PAD
    dst_pad_deg = src_pad
    src3 = jnp.concatenate([src, src_pad]).reshape(E_PAD // CHUNK, CHUNK)
    dst3 = jnp.concatenate([dst, dst_pad_prop]).reshape(E_PAD // CHUNK, CHUNK)
    dstd = jnp.concatenate([dst, dst_pad_deg]).reshape(E_PAD // CHUNK, CHUNK)
    x_pad = jnp.pad(x.astype(jnp.float32), ((0, N_PAD - N_NODES), (0, 0)))
    zeros = jnp.zeros((N_PAD, F), jnp.float32)
    b1r = b1.reshape(1, -1)
    b2r = b2.reshape(1, -1)
    b3r = b3.reshape(1, -1)

    degp = _sc_degree(dstd)
    degp_col = degp.reshape(NC, N_PAD, 1)
    g0, dinv, ca, cb = _tc_prep(x_pad, W1, W2, W3, b1r, b2r, degp_col)
    p = _sc_propagate(g0, src3, dst3, zeros)
    g1 = _tc_combine(p, g0, dinv, ca)
    p = _sc_propagate(g1, src3, dst3, zeros)
    g2 = _tc_combine(p, g1, dinv, cb)
    p = _sc_propagate(g2, src3, dst3, zeros)
    out = _tc_final(p, g2, dinv, b3r)
    return out[:N_NODES]
